# baseline (speedup 1.0000x reference)
import functools
import math

import jax
import jax.numpy as jnp
from jax.experimental import pallas as pl
from jax.experimental.pallas import tpu as pltpu

B, K, D = 4096, 128, 128
TB = 2048
RC = 64
UNROLL = 8

_LOG2E = math.log2(math.e)


@functools.lru_cache(maxsize=1)
def _gumbel_const():
    with jax.ensure_compile_time_eval():
        return jax.random.gumbel(jax.random.key(42), (B, D), jnp.float32)


def _gmm_kernel(x_ref, mu_ref, sigma_ref, g_ref, out_ref,
                p_ref, q_ref, r_ref, acc_ref):
    @pl.when(pl.program_id(0) == 0)
    def _init_coeffs():
        mu = mu_ref[...]
        sigma = sigma_ref[...]
        inv_var = 1.0 / (sigma * sigma)
        p_ref[...] = (-0.5 * _LOG2E) * inv_var
        q_ref[...] = (_LOG2E * mu) * inv_var
        r_ref[...] = ((-0.5 * _LOG2E) * (mu * mu) * inv_var
                      - jnp.log2(sigma) - 0.5 * math.log2(2 * math.pi))

    def chunk(c, _):
        x = x_ref[pl.ds(c * RC, RC), :]
        x2 = x * x

        def body(i, carry):
            accs = list(carry)
            k0 = i * UNROLL
            for u in range(UNROLL):
                pk = p_ref[pl.ds(k0 + u, 1), :]
                qk = q_ref[pl.ds(k0 + u, 1), :]
                rk = r_ref[pl.ds(k0 + u, 1), :]
                accs[u % 2] = accs[u % 2] + jnp.exp2(x2 * pk + x * qk + rk)
            return tuple(accs)

        zero = jnp.zeros((RC, D), jnp.float32)
        accs = jax.lax.fori_loop(0, K // UNROLL, body, (zero,) * 2)
        acc = accs[0] + accs[1]
        acc_ref[pl.ds(c * RC, RC), :] = acc
        return 0

    jax.lax.fori_loop(0, TB // RC, chunk, 0)

    acc = acc_ref[...]
    s = jnp.sum(acc, axis=1, keepdims=True)
    t = acc / s
    scores = jnp.log(t + 1e-30) + g_ref[...]
    samp = jnp.argmax(scores, axis=1)

    onehot = (samp[:, None]
              == jax.lax.broadcasted_iota(jnp.int32, (TB, K), 1)
              ).astype(jnp.float32)
    out_ref[...] = jax.lax.dot_general(
        onehot, mu_ref[...], (((1,), (0,)), ((), ())),
        precision=jax.lax.Precision.HIGHEST,
        preferred_element_type=jnp.float32)


def kernel(x, mu, sigma):
    g = _gumbel_const()
    grid = (B // TB,)
    return pl.pallas_call(
        _gmm_kernel,
        grid=grid,
        in_specs=[
            pl.BlockSpec((TB, D), lambda i: (i, 0)),
            pl.BlockSpec((K, D), lambda i: (0, 0)),
            pl.BlockSpec((K, D), lambda i: (0, 0)),
            pl.BlockSpec((TB, D), lambda i: (i, 0)),
        ],
        out_specs=pl.BlockSpec((TB, D), lambda i: (i, 0)),
        out_shape=jax.ShapeDtypeStruct((B, D), jnp.float32),
        scratch_shapes=[pltpu.VMEM((K, D), jnp.float32)] * 3
        + [pltpu.VMEM((TB, D), jnp.float32)],
    )(x, mu, sigma, g)


# speedup vs baseline: 1.0135x; 1.0135x over previous
import functools
import math

import jax
import jax.numpy as jnp
from jax.experimental import pallas as pl
from jax.experimental.pallas import tpu as pltpu

B, K, D = 4096, 128, 128
TB = 2048
RC = 32
UNROLL = 16

_LOG2E = math.log2(math.e)


@functools.lru_cache(maxsize=1)
def _gumbel_const():
    with jax.ensure_compile_time_eval():
        return jax.random.gumbel(jax.random.key(42), (B, D), jnp.float32)


def _gmm_kernel(x_ref, mu_ref, sigma_ref, g_ref, out_ref,
                p_ref, q_ref, r_ref, acc_ref):
    @pl.when(pl.program_id(0) == 0)
    def _init_coeffs():
        mu = mu_ref[...]
        sigma = sigma_ref[...]
        inv_var = 1.0 / (sigma * sigma)
        p_ref[...] = (-0.5 * _LOG2E) * inv_var
        q_ref[...] = (_LOG2E * mu) * inv_var
        r_ref[...] = ((-0.5 * _LOG2E) * (mu * mu) * inv_var
                      - jnp.log2(sigma) - 0.5 * math.log2(2 * math.pi))

    def chunk(c, _):
        x = x_ref[pl.ds(c * RC, RC), :]
        x2 = x * x

        def body(i, carry):
            accs = list(carry)
            k0 = i * UNROLL
            for u in range(UNROLL):
                pk = p_ref[pl.ds(k0 + u, 1), :]
                qk = q_ref[pl.ds(k0 + u, 1), :]
                rk = r_ref[pl.ds(k0 + u, 1), :]
                accs[u % 2] = accs[u % 2] + jnp.exp2(x2 * pk + x * qk + rk)
            return tuple(accs)

        zero = jnp.zeros((RC, D), jnp.float32)
        accs = jax.lax.fori_loop(0, K // UNROLL, body, (zero,) * 2)
        acc = accs[0] + accs[1]
        acc_ref[pl.ds(c * RC, RC), :] = acc
        return 0

    jax.lax.fori_loop(0, TB // RC, chunk, 0)

    acc = acc_ref[...]
    s = jnp.sum(acc, axis=1, keepdims=True)
    t = acc / s
    scores = jnp.log(t + 1e-30) + g_ref[...]
    samp = jnp.argmax(scores, axis=1)

    onehot = (samp[:, None]
              == jax.lax.broadcasted_iota(jnp.int32, (TB, K), 1)
              ).astype(jnp.float32)
    out_ref[...] = jax.lax.dot_general(
        onehot, mu_ref[...], (((1,), (0,)), ((), ())),
        precision=jax.lax.Precision.HIGHEST,
        preferred_element_type=jnp.float32)


def kernel(x, mu, sigma):
    g = _gumbel_const()
    grid = (B // TB,)
    return pl.pallas_call(
        _gmm_kernel,
        grid=grid,
        in_specs=[
            pl.BlockSpec((TB, D), lambda i: (i, 0)),
            pl.BlockSpec((K, D), lambda i: (0, 0)),
            pl.BlockSpec((K, D), lambda i: (0, 0)),
            pl.BlockSpec((TB, D), lambda i: (i, 0)),
        ],
        out_specs=pl.BlockSpec((TB, D), lambda i: (i, 0)),
        out_shape=jax.ShapeDtypeStruct((B, D), jnp.float32),
        scratch_shapes=[pltpu.VMEM((K, D), jnp.float32)] * 3
        + [pltpu.VMEM((TB, D), jnp.float32)],
    )(x, mu, sigma, g)


# speedup vs baseline: 1.0939x; 1.0793x over previous
import functools
import math

import jax
import jax.numpy as jnp
from jax.experimental import pallas as pl
from jax.experimental.pallas import tpu as pltpu

B, K, D = 4096, 128, 128
TB = 2048
RC = 64
UNROLL = 16

_LOG2E = math.log2(math.e)


@functools.lru_cache(maxsize=1)
def _gumbel_const():
    with jax.ensure_compile_time_eval():
        return jax.random.gumbel(jax.random.key(42), (B, D), jnp.float32)


def _gmm_kernel(x_ref, mu_ref, sigma_ref, g_ref, out_ref,
                p_ref, q_ref, r_ref, acc_ref):
    @pl.when(pl.program_id(0) == 0)
    def _init_coeffs():
        mu = mu_ref[...]
        sigma = sigma_ref[...]
        inv_var = 1.0 / (sigma * sigma)
        p_ref[...] = (-0.5 * _LOG2E) * inv_var
        q_ref[...] = mu
        r_ref[...] = -jnp.log2(sigma) - 0.5 * math.log2(2 * math.pi)

    def chunk(c, _):
        x = x_ref[pl.ds(c * RC, RC), :]

        def body(i, carry):
            accs = list(carry)
            k0 = i * UNROLL
            for u in range(UNROLL):
                pk = p_ref[pl.ds(k0 + u, 1), :]
                qk = q_ref[pl.ds(k0 + u, 1), :]
                rk = r_ref[pl.ds(k0 + u, 1), :]
                t = x - qk
                accs[u % 2] = accs[u % 2] + jnp.exp2((t * t) * pk + rk)
            return tuple(accs)

        zero = jnp.zeros((RC, D), jnp.float32)
        accs = jax.lax.fori_loop(0, K // UNROLL, body, (zero,) * 2)
        acc = accs[0] + accs[1]
        acc_ref[pl.ds(c * RC, RC), :] = acc
        return 0

    jax.lax.fori_loop(0, TB // RC, chunk, 0)

    acc = acc_ref[...]
    s = jnp.sum(acc, axis=1, keepdims=True)
    t = acc / s
    scores = jnp.log(t + 1e-30) + g_ref[...]
    samp = jnp.argmax(scores, axis=1)

    onehot = (samp[:, None]
              == jax.lax.broadcasted_iota(jnp.int32, (TB, K), 1)
              ).astype(jnp.float32)
    out_ref[...] = jax.lax.dot_general(
        onehot, mu_ref[...], (((1,), (0,)), ((), ())),
        precision=jax.lax.Precision.HIGHEST,
        preferred_element_type=jnp.float32)


def kernel(x, mu, sigma):
    g = _gumbel_const()
    grid = (B // TB,)
    return pl.pallas_call(
        _gmm_kernel,
        grid=grid,
        in_specs=[
            pl.BlockSpec((TB, D), lambda i: (i, 0)),
            pl.BlockSpec((K, D), lambda i: (0, 0)),
            pl.BlockSpec((K, D), lambda i: (0, 0)),
            pl.BlockSpec((TB, D), lambda i: (i, 0)),
        ],
        out_specs=pl.BlockSpec((TB, D), lambda i: (i, 0)),
        out_shape=jax.ShapeDtypeStruct((B, D), jnp.float32),
        scratch_shapes=[pltpu.VMEM((K, D), jnp.float32)] * 3
        + [pltpu.VMEM((TB, D), jnp.float32)],
    )(x, mu, sigma, g)
